# baseline (speedup 1.0000x reference)
import functools
import math

import jax
import jax.numpy as jnp
from jax import lax
from jax.experimental import pallas as pl
from jax.experimental.pallas import tpu as pltpu
from jax.experimental.pallas import tpu_sc as plsc

N_Q = 10000
N_S = 10000
NN = 32
IN_DIM = 128
OUT_DIM = 128
KL = 27
RADIUS = 0.2
KP_EXTENT = 2.0 * RADIUS / 2.0 / math.sqrt(3.0)
ROWS = N_Q * NN

_NC = 2
_NSC = 16
_NW = _NC * _NSC
_NS_PAD = 10008

_QT = 320
_NQP = _NW * _QT
_NQ_PAD = 10248
_CAPT = 64
_CAPQ = _NW * _CAPT
_CROWS = _CAPQ * NN
_THR = (RADIUS + KP_EXTENT) ** 2 * (1.0 + 1e-5) + 1e-7


def _sc_filter_body(idx_hbm, xpad_hbm, qx_hbm, qy_hbm, qz_hbm,
                    sx_hbm, sy_hbm, sz_hbm, zz_hbm,
                    xgc_out, relc_out, slot_out, cnt_out,
                    qx_v, qy_v, qz_v, sx_v, sy_v, sz_v,
                    idx_sl, pidx_v, qid_v, slot_sl, xr_v, rel_v, cnt_v, sem):
    t = lax.axis_index("s") * _NC + lax.axis_index("c")
    lanes = lax.iota(jnp.int32, 16)

    pltpu.sync_copy(qx_hbm, qx_v)
    pltpu.sync_copy(qy_hbm, qy_v)
    pltpu.sync_copy(qz_hbm, qz_v)
    pltpu.sync_copy(sx_hbm, sx_v)
    pltpu.sync_copy(sy_hbm, sy_v)
    pltpu.sync_copy(sz_hbm, sz_v)
    pltpu.sync_copy(idx_hbm.at[pl.ds(t * (_QT * NN), _QT * NN)], idx_sl)
    pltpu.sync_copy(zz_hbm, rel_v)

    def fill(m, c):
        pidx_v[pl.ds(m * 16, 16)] = jnp.full((16,), N_S, jnp.int32)
        return c
    lax.fori_loop(0, (_CAPT * NN) // 16, fill, 0)
    for m in range(8):
        qid_v[pl.ds(m * 16, 16)] = jnp.zeros((16,), jnp.int32)

    def fbody(j, cnt):
        qloc = j * 16 + lanes
        qglob = t * _QT + qloc
        qxv = plsc.load_gather(qx_v, [qglob])
        qyv = plsc.load_gather(qy_v, [qglob])
        qzv = plsc.load_gather(qz_v, [qglob])
        act = jnp.zeros((16,), jnp.bool_)
        for k in range(NN):
            ixk = plsc.load_gather(idx_sl, [qloc * NN + k])
            dx = plsc.load_gather(sx_v, [ixk]) - qxv
            dy = plsc.load_gather(sy_v, [ixk]) - qyv
            dz = plsc.load_gather(sz_v, [ixk]) - qzv
            act = act | (dx * dx + dy * dy + dz * dz < _THR)
        acti = jnp.where(act, 1, 0).astype(jnp.int32)
        pref = plsc.cumsum(acti)
        lraw = cnt + pref - acti
        lsafe = jnp.minimum(lraw, _CAPT - 1)
        slot_sl[pl.ds(j * 16, 16)] = jnp.where(act, t * _CAPT + lsafe, _CAPQ)
        plsc.store_scatter(qid_v, [lsafe], qglob, mask=act)
        for k in range(NN):
            vals = plsc.load_gather(idx_sl, [qloc * NN + k])
            plsc.store_scatter(pidx_v, [lsafe * NN + k], vals, mask=act)
        return cnt + jnp.sum(acti)
    with jax.named_scope("flt_phase1"):
        cnt = lax.fori_loop(0, _QT // 16, fbody, jnp.int32(0))

    def gbody(c, carry):
        base = c * 128
        cp = pltpu.async_copy(xpad_hbm.at[pidx_v.at[pl.ds(base, 128)]],
                              xr_v, sem)
        for j2 in range(8):
            rows16 = j2 * 16 + lanes
            pv = pidx_v[pl.ds(base + j2 * 16, 16)]
            lslotv = (base + j2 * 16 + lanes) >> 5
            qidv = plsc.load_gather(qid_v, [lslotv])
            rx = plsc.load_gather(sx_v, [pv]) - plsc.load_gather(qx_v, [qidv])
            ry = plsc.load_gather(sy_v, [pv]) - plsc.load_gather(qy_v, [qidv])
            rz = plsc.load_gather(sz_v, [pv]) - plsc.load_gather(qz_v, [qidv])
            zero = jnp.zeros((16,), jnp.int32)
            plsc.store_scatter(rel_v, [rows16, zero], rx)
            plsc.store_scatter(rel_v, [rows16, zero + 1], ry)
            plsc.store_scatter(rel_v, [rows16, zero + 2], rz)
        cp.wait()
        out_base = t * (_CAPT * NN) + base
        pltpu.sync_copy(xr_v, xgc_out.at[pl.ds(out_base, 128)])
        pltpu.sync_copy(rel_v, relc_out.at[pl.ds(out_base, 128)])
        return carry
    with jax.named_scope("flt_phase2"):
        lax.fori_loop(0, (_CAPT * NN) // 128, gbody, 0)

    pltpu.sync_copy(slot_sl, slot_out.at[pl.ds(t * _QT, _QT)])
    cnt_v[...] = jnp.broadcast_to(cnt, (16,)).astype(jnp.int32)
    pltpu.sync_copy(cnt_v, cnt_out.at[pl.ds(t * 16, 16)])


@functools.lru_cache(maxsize=1)
def _sc_filter():
    return pl.kernel(
        _sc_filter_body,
        out_type=(jax.ShapeDtypeStruct((_CROWS, IN_DIM), jnp.float32),
                  jax.ShapeDtypeStruct((_CROWS, 16), jnp.float32),
                  jax.ShapeDtypeStruct((_NQP,), jnp.int32),
                  jax.ShapeDtypeStruct((_NW * 16,), jnp.int32)),
        mesh=plsc.VectorSubcoreMesh(core_axis_name="c", subcore_axis_name="s"),
        compiler_params=pltpu.CompilerParams(needs_layout_passes=False),
        scratch_types=[
            pltpu.VMEM((_NQ_PAD,), jnp.float32),
            pltpu.VMEM((_NQ_PAD,), jnp.float32),
            pltpu.VMEM((_NQ_PAD,), jnp.float32),
            pltpu.VMEM((_NS_PAD,), jnp.float32),
            pltpu.VMEM((_NS_PAD,), jnp.float32),
            pltpu.VMEM((_NS_PAD,), jnp.float32),
            pltpu.VMEM((_QT * NN,), jnp.int32),
            pltpu.VMEM((_CAPT * NN,), jnp.int32),
            pltpu.VMEM((128,), jnp.int32),
            pltpu.VMEM((_QT,), jnp.int32),
            pltpu.VMEM((128, IN_DIM), jnp.float32),
            pltpu.VMEM((128, 16), jnp.float32),
            pltpu.VMEM((16,), jnp.int32),
            pltpu.SemaphoreType.DMA,
        ],
    )


def _sc_scatter_body(slot_hbm, fxc_hbm, out_hbm, slot_v, fr_v, sem):
    t = lax.axis_index("s") * _NC + lax.axis_index("c")
    pltpu.sync_copy(slot_hbm.at[pl.ds(t * _QT, _QT)], slot_v)

    def body(m, carry):
        pltpu.async_copy(fxc_hbm.at[slot_v.at[pl.ds(m * 64, 64)]],
                         fr_v, sem).wait()
        pltpu.sync_copy(fr_v, out_hbm.at[pl.ds(t * _QT + m * 64, 64)])
        return carry
    lax.fori_loop(0, _QT // 64, body, 0)


@functools.lru_cache(maxsize=1)
def _sc_scatter():
    return pl.kernel(
        _sc_scatter_body,
        out_type=jax.ShapeDtypeStruct((_NQP, OUT_DIM), jnp.float32),
        mesh=plsc.VectorSubcoreMesh(core_axis_name="c", subcore_axis_name="s"),
        compiler_params=pltpu.CompilerParams(needs_layout_passes=False),
        scratch_types=[
            pltpu.VMEM((_QT,), jnp.int32),
            pltpu.VMEM((64, OUT_DIM), jnp.float32),
            pltpu.SemaphoreType.DMA,
        ],
    )


_CH = 192
_SPLITS = ((0, 128), (128, 64))
_NITER = 54
ROWS_PAD = _NW * _NITER * _CH


def _sc_gather_body(idx_hbm, xpad_hbm, sx_hbm, sy_hbm, sz_hbm, zz_hbm,
                    xg_out, pos_out,
                    idx0, idx1, xr0, xr1, pr0, pr1, sx_v, sy_v, sz_v,
                    sg0, sg1, sw0, sw1):
    wid = lax.axis_index("s") * _NC + lax.axis_index("c")
    pltpu.sync_copy(sx_hbm, sx_v)
    pltpu.sync_copy(sy_hbm, sy_v)
    pltpu.sync_copy(sz_hbm, sz_v)
    pltpu.sync_copy(zz_hbm, pr0)
    pltpu.sync_copy(zz_hbm, pr1)

    def start(c, idx_b, xr_b, sg):
        pltpu.sync_copy(idx_hbm.at[pl.ds(c * _CH, _CH)], idx_b)
        for off, ln in _SPLITS:
            pltpu.async_copy(
                xpad_hbm.at[idx_b.at[pl.ds(off, ln)]],
                xr_b.at[pl.ds(off, ln)], sg)

    def assemble_pos(idx_b, pr_b):
        lanes = lax.iota(jnp.int32, 16)
        for j in range(_CH // 16):
            ii = idx_b[pl.ds(j * 16, 16)]
            rows = lanes + (j * 16)
            px = plsc.load_gather(sx_v, [ii])
            py = plsc.load_gather(sy_v, [ii])
            pz = plsc.load_gather(sz_v, [ii])
            plsc.store_scatter(pr_b, [rows, jnp.full((16,), 0, jnp.int32)], px)
            plsc.store_scatter(pr_b, [rows, jnp.full((16,), 1, jnp.int32)], py)
            plsc.store_scatter(pr_b, [rows, jnp.full((16,), 2, jnp.int32)], pz)

    def drain_gather(xr_b, sg):
        for off, ln in _SPLITS:
            pltpu.make_async_copy(
                xpad_hbm.at[idx0.at[pl.ds(off, ln)]],
                xr_b.at[pl.ds(off, ln)], sg).wait()

    def fire_writeout(c, xr_b, pr_b, sw):
        base = c * _CH
        pltpu.async_copy(xr_b, xg_out.at[pl.ds(base, _CH)], sw)
        pltpu.async_copy(pr_b, pos_out.at[pl.ds(base, _CH)], sw)

    def drain_writeout(xr_b, pr_b, sw):
        pltpu.make_async_copy(xr_b, xg_out.at[pl.ds(0, _CH)], sw).wait()
        pltpu.make_async_copy(pr_b, pos_out.at[pl.ds(0, _CH)], sw).wait()

    def body(j, carry):
        c0 = wid + _NW * (2 * j)
        c1 = wid + _NW * (2 * j + 1)

        @pl.when(j > 0)
        def _():
            drain_writeout(xr0, pr0, sw0)
        start(c0, idx0, xr0, sg0)

        @pl.when(j > 0)
        def _():
            drain_writeout(xr1, pr1, sw1)
        start(c1, idx1, xr1, sg1)

        assemble_pos(idx0, pr0)
        drain_gather(xr0, sg0)
        fire_writeout(c0, xr0, pr0, sw0)

        assemble_pos(idx1, pr1)
        drain_gather(xr1, sg1)
        fire_writeout(c1, xr1, pr1, sw1)
        return carry

    lax.fori_loop(0, _NITER // 2, body, 0)
    drain_writeout(xr0, pr0, sw0)
    drain_writeout(xr1, pr1, sw1)


@functools.lru_cache(maxsize=1)
def _sc_gather():
    return pl.kernel(
        _sc_gather_body,
        out_type=(jax.ShapeDtypeStruct((ROWS_PAD, IN_DIM), jnp.float32),
                  jax.ShapeDtypeStruct((ROWS_PAD, 16), jnp.float32)),
        mesh=plsc.VectorSubcoreMesh(core_axis_name="c", subcore_axis_name="s"),
        compiler_params=pltpu.CompilerParams(needs_layout_passes=False),
        scratch_types=[
            pltpu.VMEM((_CH,), jnp.int32),
            pltpu.VMEM((_CH,), jnp.int32),
            pltpu.VMEM((_CH, IN_DIM), jnp.float32),
            pltpu.VMEM((_CH, IN_DIM), jnp.float32),
            pltpu.VMEM((_CH, 16), jnp.float32),
            pltpu.VMEM((_CH, 16), jnp.float32),
            pltpu.VMEM((_NS_PAD,), jnp.float32),
            pltpu.VMEM((_NS_PAD,), jnp.float32),
            pltpu.VMEM((_NS_PAD,), jnp.float32),
            pltpu.SemaphoreType.DMA,
            pltpu.SemaphoreType.DMA,
            pltpu.SemaphoreType.DMA,
            pltpu.SemaphoreType.DMA,
        ],
    )


def _conv_math(rel, kpt_ref, w_ref, xg, qb):
    kpt = kpt_ref[...]
    dots = jnp.dot(rel, kpt, preferred_element_type=jnp.float32,
                   precision=jax.lax.Precision.HIGHEST)
    kn = jnp.sum(kpt * kpt, axis=0, keepdims=True)
    sqn = jnp.sum(rel * rel, axis=1, keepdims=True)
    sq = jnp.maximum(sqn + kn - 2.0 * dots, 0.0)
    aw = jnp.maximum(1.0 - jnp.sqrt(sq) * (1.0 / KP_EXTENT), 0.0)
    aw3 = aw.astype(jnp.bfloat16).reshape(qb, NN, 32)
    acc = jnp.zeros((qb, OUT_DIM), dtype=jnp.float32)
    for l in range(KL):
        zl = aw3[:, :, l:l + 1] * xg
        zsum = jnp.sum(zl, axis=1)
        acc = acc + jnp.dot(zsum, w_ref[l],
                            preferred_element_type=jnp.float32)
    return jnp.maximum(acc, 0.0)


_QB = 200
_PB = _QB * NN
_NBLK = N_Q // _QB


def _conv_body(qrep_ref, pos_ref, xg_ref, kpt_ref, w_ref, out_ref):
    rel = pos_ref[...] - qrep_ref[...]
    xg = xg_ref[...].astype(jnp.bfloat16)
    out_ref[...] = _conv_math(rel, kpt_ref, w_ref, xg, _QB)


def _conv_call(qrep, pos, xg3, kpt, weights, interpret=False):
    return pl.pallas_call(
        _conv_body,
        grid=(_NBLK,),
        in_specs=[
            pl.BlockSpec((_PB, 16), lambda i: (i, 0)),
            pl.BlockSpec((_PB, 16), lambda i: (i, 0)),
            pl.BlockSpec((_QB, NN, IN_DIM), lambda i: (i, 0, 0)),
            pl.BlockSpec((16, 32), lambda i: (0, 0)),
            pl.BlockSpec((KL, IN_DIM, OUT_DIM), lambda i: (0, 0, 0)),
        ],
        out_specs=pl.BlockSpec((_QB, OUT_DIM), lambda i: (i, 0)),
        out_shape=jax.ShapeDtypeStruct((N_Q, OUT_DIM), jnp.float32),
        compiler_params=pltpu.CompilerParams(
            dimension_semantics=("arbitrary",)),
        interpret=interpret,
    )(qrep, pos, xg3, kpt, weights)


_QBC = 256
_NBLKC = _CAPQ // _QBC


def _conv_body_c(rel_ref, xg_ref, kpt_ref, w_ref, out_ref):
    rel = rel_ref[...]
    xg = xg_ref[...].astype(jnp.bfloat16)
    out_ref[...] = _conv_math(rel, kpt_ref, w_ref, xg, _QBC)


def _conv_call_c(relc, xgc3, kpt, weights):
    return pl.pallas_call(
        _conv_body_c,
        grid=(_NBLKC,),
        in_specs=[
            pl.BlockSpec((_QBC * NN, 16), lambda i: (i, 0)),
            pl.BlockSpec((_QBC, NN, IN_DIM), lambda i: (i, 0, 0)),
            pl.BlockSpec((16, 32), lambda i: (0, 0)),
            pl.BlockSpec((KL, IN_DIM, OUT_DIM), lambda i: (0, 0, 0)),
        ],
        out_specs=pl.BlockSpec((_QBC, OUT_DIM), lambda i: (i, 0)),
        out_shape=jax.ShapeDtypeStruct((_CAPQ, OUT_DIM), jnp.float32),
        compiler_params=pltpu.CompilerParams(
            dimension_semantics=("arbitrary",)),
    )(relc, xgc3, kpt, weights)


def kernel(q_pts, s_pts, neighb_inds, x, weights, kernel_points):
    idx = (neighb_inds.astype(jnp.int32) % (N_S + 1)).reshape(-1)
    idx_pad = jnp.concatenate(
        [idx, jnp.zeros((ROWS_PAD - ROWS,), jnp.int32)])
    x_pad = jnp.concatenate([x, jnp.zeros((1, IN_DIM), x.dtype)], axis=0)
    tail = jnp.concatenate([jnp.full((1,), 1.0e6, s_pts.dtype),
                            jnp.zeros((_NS_PAD - N_S - 1,), s_pts.dtype)])
    sx = jnp.concatenate([s_pts[:, 0], tail])
    sy = jnp.concatenate([s_pts[:, 1], tail])
    sz = jnp.concatenate([s_pts[:, 2], tail])
    qtail = jnp.full((_NQ_PAD - N_Q,), 1.0e6, q_pts.dtype)
    qx = jnp.concatenate([q_pts[:, 0], qtail])
    qy = jnp.concatenate([q_pts[:, 1], qtail])
    qz = jnp.concatenate([q_pts[:, 2], qtail])
    zz = jnp.zeros((128, 16), jnp.float32)
    kpt = jnp.pad(kernel_points, ((0, 5), (0, 13))).T
    w_bf = weights.astype(jnp.bfloat16)

    xgc, relc, slot_map, cnts = _sc_filter()(
        idx_pad, x_pad, qx, qy, qz, sx, sy, sz, zz)
    overflow = jnp.any(cnts[0::16] > _CAPT)

    def sparse_path(op):
        xgc_, relc_, slot_map_ = op
        fxc = _conv_call_c(relc_, xgc_.reshape(_CAPQ, NN, IN_DIM), kpt, w_bf)
        fxc_pad = jnp.concatenate(
            [fxc, jnp.zeros((8, OUT_DIM), jnp.float32)], axis=0)
        fxp = _sc_scatter()(slot_map_, fxc_pad)
        return fxp[:N_Q]

    def dense_path(op):
        zzc = jnp.zeros((_CH, 16), jnp.float32)
        q16 = jnp.pad(q_pts, ((0, 0), (0, 13)))
        qrep = jnp.broadcast_to(
            q16[:, None, :], (N_Q, NN, 16)).reshape(ROWS, 16)
        xg, pos = _sc_gather()(idx_pad, x_pad, sx, sy, sz, zzc)
        xg3 = xg.reshape(ROWS_PAD // NN, NN, IN_DIM)
        return _conv_call(qrep, pos, xg3, kpt, w_bf)

    return lax.cond(overflow, dense_path, sparse_path, (xgc, relc, slot_map))


# speedup vs baseline: 1.0032x; 1.0032x over previous
import functools
import math

import jax
import jax.numpy as jnp
from jax import lax
from jax.experimental import pallas as pl
from jax.experimental.pallas import tpu as pltpu
from jax.experimental.pallas import tpu_sc as plsc

N_Q = 10000
N_S = 10000
NN = 32
IN_DIM = 128
OUT_DIM = 128
KL = 27
RADIUS = 0.2
KP_EXTENT = 2.0 * RADIUS / 2.0 / math.sqrt(3.0)
ROWS = N_Q * NN

_NC = 2
_NSC = 16
_NW = _NC * _NSC
_NS_PAD = 10008

_QT = 320
_NQP = _NW * _QT
_NQ_PAD = 10248
_CAPT = 64
_CAPQ = _NW * _CAPT
_CROWS = _CAPQ * NN
_THR = (RADIUS + KP_EXTENT) ** 2 * (1.0 + 1e-5) + 1e-7


def _sc_filter_body(idx_hbm, xpad_hbm, qx_hbm, qy_hbm, qz_hbm,
                    sx_hbm, sy_hbm, sz_hbm, zz_hbm,
                    xgc_out, relc_out, slot_out, cnt_out,
                    qx_v, qy_v, qz_v, sx_v, sy_v, sz_v,
                    idx_sl, pidx_v, qid_v, slot_sl, xr0, xr1, rel_v, cnt_v,
                    pidx_s0, pidx_s1, semA, semB):
    t = lax.axis_index("s") * _NC + lax.axis_index("c")
    lanes = lax.iota(jnp.int32, 16)
    xr_v = [xr0, xr1]
    sems = [semA, semB]

    pltpu.sync_copy(qx_hbm, qx_v)
    pltpu.sync_copy(qy_hbm, qy_v)
    pltpu.sync_copy(qz_hbm, qz_v)
    pltpu.sync_copy(sx_hbm, sx_v)
    pltpu.sync_copy(sy_hbm, sy_v)
    pltpu.sync_copy(sz_hbm, sz_v)
    pltpu.sync_copy(idx_hbm.at[pl.ds(t * (_QT * NN), _QT * NN)], idx_sl)
    pltpu.sync_copy(zz_hbm.at[pl.ds(0, 64)], rel_v)

    def fill(m, c):
        pidx_v[pl.ds(m * 16, 16)] = jnp.full((16,), N_S, jnp.int32)
        return c
    lax.fori_loop(0, (_CAPT * NN) // 16, fill, 0)
    for m in range(8):
        qid_v[pl.ds(m * 16, 16)] = jnp.zeros((16,), jnp.int32)

    def fbody(j, cnt):
        qloc = j * 16 + lanes
        qglob = t * _QT + qloc
        qxv = plsc.load_gather(qx_v, [qglob])
        qyv = plsc.load_gather(qy_v, [qglob])
        qzv = plsc.load_gather(qz_v, [qglob])
        act = jnp.zeros((16,), jnp.bool_)
        for k in range(NN):
            ixk = plsc.load_gather(idx_sl, [qloc * NN + k])
            dx = plsc.load_gather(sx_v, [ixk]) - qxv
            dy = plsc.load_gather(sy_v, [ixk]) - qyv
            dz = plsc.load_gather(sz_v, [ixk]) - qzv
            act = act | (dx * dx + dy * dy + dz * dz < _THR)
        acti = jnp.where(act, 1, 0).astype(jnp.int32)
        pref = plsc.cumsum(acti)
        lraw = cnt + pref - acti
        lsafe = jnp.minimum(lraw, _CAPT - 1)
        slot_sl[pl.ds(j * 16, 16)] = jnp.where(act, t * _CAPT + lsafe, _CAPQ)
        plsc.store_scatter(qid_v, [lsafe], qglob, mask=act)
        for k in range(NN):
            vals = plsc.load_gather(idx_sl, [qloc * NN + k])
            plsc.store_scatter(pidx_v, [lsafe * NN + k], vals, mask=act)
        return cnt + jnp.sum(acti)
    with jax.named_scope("flt_phase1"):
        cnt = lax.fori_loop(0, _QT // 16, fbody, jnp.int32(0))

    _NCH = (_CAPT * NN) // 64
    pidx_sm = [pidx_s0, pidx_s1]

    def _stage_idx(c, b):
        for k4 in range(4):
            pidx_sm[b][pl.ds(k4 * 16, 16)] = \
                pidx_v[pl.ds(c * 64 + k4 * 16, 16)]

    with jax.named_scope("flt_phase2"):
        cps = [None, None]
        _stage_idx(0, 0)
        cps[0] = pltpu.async_copy(
            xpad_hbm.at[pidx_sm[0]], xr_v[0], sems[0])
        for c in range(_NCH):
            b = c % 2
            if c + 1 < _NCH:
                nb = (c + 1) % 2
                _stage_idx(c + 1, nb)
                cps[nb] = pltpu.async_copy(
                    xpad_hbm.at[pidx_sm[nb]], xr_v[nb], sems[nb])
            base = c * 64
            for j2 in range(4):
                rows16 = j2 * 16 + lanes
                pv = pidx_v[pl.ds(base + j2 * 16, 16)]
                lslotv = jnp.full((16,), (base + j2 * 16) // NN, jnp.int32)
                qidv = plsc.load_gather(qid_v, [lslotv])
                rx = plsc.load_gather(sx_v, [pv]) - plsc.load_gather(qx_v, [qidv])
                ry = plsc.load_gather(sy_v, [pv]) - plsc.load_gather(qy_v, [qidv])
                rz = plsc.load_gather(sz_v, [pv]) - plsc.load_gather(qz_v, [qidv])
                zero = jnp.zeros((16,), jnp.int32)
                plsc.store_scatter(rel_v, [rows16, zero], rx)
                plsc.store_scatter(rel_v, [rows16, zero + 1], ry)
                plsc.store_scatter(rel_v, [rows16, zero + 2], rz)
            cps[b].wait()
            out_base = t * (_CAPT * NN) + base
            pltpu.sync_copy(xr_v[b], xgc_out.at[pl.ds(out_base, 64)])
            pltpu.sync_copy(rel_v, relc_out.at[pl.ds(out_base, 64)])

    pltpu.sync_copy(slot_sl, slot_out.at[pl.ds(t * _QT, _QT)])
    cnt_v[...] = jnp.broadcast_to(cnt, (16,)).astype(jnp.int32)
    pltpu.sync_copy(cnt_v, cnt_out.at[pl.ds(t * 16, 16)])


@functools.lru_cache(maxsize=1)
def _sc_filter():
    return pl.kernel(
        _sc_filter_body,
        out_type=(jax.ShapeDtypeStruct((_CROWS, IN_DIM), jnp.float32),
                  jax.ShapeDtypeStruct((_CROWS, 16), jnp.float32),
                  jax.ShapeDtypeStruct((_NQP,), jnp.int32),
                  jax.ShapeDtypeStruct((_NW * 16,), jnp.int32)),
        mesh=plsc.VectorSubcoreMesh(core_axis_name="c", subcore_axis_name="s"),
        compiler_params=pltpu.CompilerParams(needs_layout_passes=False),
        scratch_types=[
            pltpu.VMEM((_NQ_PAD,), jnp.float32),
            pltpu.VMEM((_NQ_PAD,), jnp.float32),
            pltpu.VMEM((_NQ_PAD,), jnp.float32),
            pltpu.VMEM((_NS_PAD,), jnp.float32),
            pltpu.VMEM((_NS_PAD,), jnp.float32),
            pltpu.VMEM((_NS_PAD,), jnp.float32),
            pltpu.VMEM((_QT * NN,), jnp.int32),
            pltpu.VMEM((_CAPT * NN,), jnp.int32),
            pltpu.VMEM((128,), jnp.int32),
            pltpu.VMEM((_QT,), jnp.int32),
            pltpu.VMEM((64, IN_DIM), jnp.float32),
            pltpu.VMEM((64, IN_DIM), jnp.float32),
            pltpu.VMEM((64, 16), jnp.float32),
            pltpu.VMEM((16,), jnp.int32),
            pltpu.VMEM((64,), jnp.int32),
            pltpu.VMEM((64,), jnp.int32),
            pltpu.SemaphoreType.DMA,
            pltpu.SemaphoreType.DMA,
        ],
    )


def _sc_scatter_body(slot_hbm, fxc_hbm, out_hbm, slot_v, fr_v,
                     s0, s1, s2, s3, s4, sem):
    t = lax.axis_index("s") * _NC + lax.axis_index("c")
    pltpu.sync_copy(slot_hbm.at[pl.ds(t * _QT, _QT)], slot_v)
    sm = [s0, s1, s2, s3, s4]
    cps = []
    for m in range(_QT // 64):
        for k4 in range(4):
            sm[m][pl.ds(k4 * 16, 16)] = slot_v[pl.ds(m * 64 + k4 * 16, 16)]
        cps.append(pltpu.async_copy(
            fxc_hbm.at[sm[m]], fr_v.at[pl.ds(m * 64, 64)], sem))
    for cp in cps:
        cp.wait()
    pltpu.sync_copy(fr_v, out_hbm.at[pl.ds(t * _QT, _QT)])


@functools.lru_cache(maxsize=1)
def _sc_scatter():
    return pl.kernel(
        _sc_scatter_body,
        out_type=jax.ShapeDtypeStruct((_NQP, OUT_DIM), jnp.float32),
        mesh=plsc.VectorSubcoreMesh(core_axis_name="c", subcore_axis_name="s"),
        compiler_params=pltpu.CompilerParams(needs_layout_passes=False),
        scratch_types=[
            pltpu.VMEM((_QT,), jnp.int32),
            pltpu.VMEM((_QT, OUT_DIM), jnp.float32),
            pltpu.VMEM((64,), jnp.int32),
            pltpu.VMEM((64,), jnp.int32),
            pltpu.VMEM((64,), jnp.int32),
            pltpu.VMEM((64,), jnp.int32),
            pltpu.VMEM((64,), jnp.int32),
            pltpu.SemaphoreType.DMA,
        ],
    )


_CH = 192
_SPLITS = ((0, 128), (128, 64))
_NITER = 54
ROWS_PAD = _NW * _NITER * _CH


def _sc_gather_body(idx_hbm, xpad_hbm, sx_hbm, sy_hbm, sz_hbm, zz_hbm,
                    xg_out, pos_out,
                    idx0, idx1, xr0, xr1, pr0, pr1, sx_v, sy_v, sz_v,
                    sg0, sg1, sw0, sw1):
    wid = lax.axis_index("s") * _NC + lax.axis_index("c")
    pltpu.sync_copy(sx_hbm, sx_v)
    pltpu.sync_copy(sy_hbm, sy_v)
    pltpu.sync_copy(sz_hbm, sz_v)
    pltpu.sync_copy(zz_hbm, pr0)
    pltpu.sync_copy(zz_hbm, pr1)

    def start(c, idx_b, xr_b, sg):
        pltpu.sync_copy(idx_hbm.at[pl.ds(c * _CH, _CH)], idx_b)
        for off, ln in _SPLITS:
            pltpu.async_copy(
                xpad_hbm.at[idx_b.at[pl.ds(off, ln)]],
                xr_b.at[pl.ds(off, ln)], sg)

    def assemble_pos(idx_b, pr_b):
        lanes = lax.iota(jnp.int32, 16)
        for j in range(_CH // 16):
            ii = idx_b[pl.ds(j * 16, 16)]
            rows = lanes + (j * 16)
            px = plsc.load_gather(sx_v, [ii])
            py = plsc.load_gather(sy_v, [ii])
            pz = plsc.load_gather(sz_v, [ii])
            plsc.store_scatter(pr_b, [rows, jnp.full((16,), 0, jnp.int32)], px)
            plsc.store_scatter(pr_b, [rows, jnp.full((16,), 1, jnp.int32)], py)
            plsc.store_scatter(pr_b, [rows, jnp.full((16,), 2, jnp.int32)], pz)

    def drain_gather(xr_b, sg):
        for off, ln in _SPLITS:
            pltpu.make_async_copy(
                xpad_hbm.at[idx0.at[pl.ds(off, ln)]],
                xr_b.at[pl.ds(off, ln)], sg).wait()

    def fire_writeout(c, xr_b, pr_b, sw):
        base = c * _CH
        pltpu.async_copy(xr_b, xg_out.at[pl.ds(base, _CH)], sw)
        pltpu.async_copy(pr_b, pos_out.at[pl.ds(base, _CH)], sw)

    def drain_writeout(xr_b, pr_b, sw):
        pltpu.make_async_copy(xr_b, xg_out.at[pl.ds(0, _CH)], sw).wait()
        pltpu.make_async_copy(pr_b, pos_out.at[pl.ds(0, _CH)], sw).wait()

    def body(j, carry):
        c0 = wid + _NW * (2 * j)
        c1 = wid + _NW * (2 * j + 1)

        @pl.when(j > 0)
        def _():
            drain_writeout(xr0, pr0, sw0)
        start(c0, idx0, xr0, sg0)

        @pl.when(j > 0)
        def _():
            drain_writeout(xr1, pr1, sw1)
        start(c1, idx1, xr1, sg1)

        assemble_pos(idx0, pr0)
        drain_gather(xr0, sg0)
        fire_writeout(c0, xr0, pr0, sw0)

        assemble_pos(idx1, pr1)
        drain_gather(xr1, sg1)
        fire_writeout(c1, xr1, pr1, sw1)
        return carry

    lax.fori_loop(0, _NITER // 2, body, 0)
    drain_writeout(xr0, pr0, sw0)
    drain_writeout(xr1, pr1, sw1)


@functools.lru_cache(maxsize=1)
def _sc_gather():
    return pl.kernel(
        _sc_gather_body,
        out_type=(jax.ShapeDtypeStruct((ROWS_PAD, IN_DIM), jnp.float32),
                  jax.ShapeDtypeStruct((ROWS_PAD, 16), jnp.float32)),
        mesh=plsc.VectorSubcoreMesh(core_axis_name="c", subcore_axis_name="s"),
        compiler_params=pltpu.CompilerParams(needs_layout_passes=False),
        scratch_types=[
            pltpu.VMEM((_CH,), jnp.int32),
            pltpu.VMEM((_CH,), jnp.int32),
            pltpu.VMEM((_CH, IN_DIM), jnp.float32),
            pltpu.VMEM((_CH, IN_DIM), jnp.float32),
            pltpu.VMEM((_CH, 16), jnp.float32),
            pltpu.VMEM((_CH, 16), jnp.float32),
            pltpu.VMEM((_NS_PAD,), jnp.float32),
            pltpu.VMEM((_NS_PAD,), jnp.float32),
            pltpu.VMEM((_NS_PAD,), jnp.float32),
            pltpu.SemaphoreType.DMA,
            pltpu.SemaphoreType.DMA,
            pltpu.SemaphoreType.DMA,
            pltpu.SemaphoreType.DMA,
        ],
    )


def _conv_math(rel, kpt_ref, w_ref, xg, qb):
    kpt = kpt_ref[...]
    dots = jnp.dot(rel, kpt, preferred_element_type=jnp.float32,
                   precision=jax.lax.Precision.HIGHEST)
    kn = jnp.sum(kpt * kpt, axis=0, keepdims=True)
    sqn = jnp.sum(rel * rel, axis=1, keepdims=True)
    sq = jnp.maximum(sqn + kn - 2.0 * dots, 0.0)
    aw = jnp.maximum(1.0 - jnp.sqrt(sq) * (1.0 / KP_EXTENT), 0.0)
    aw3 = aw.astype(jnp.bfloat16).reshape(qb, NN, 32)
    acc = jnp.zeros((qb, OUT_DIM), dtype=jnp.float32)
    for l in range(KL):
        zl = aw3[:, :, l:l + 1] * xg
        zsum = jnp.sum(zl, axis=1)
        acc = acc + jnp.dot(zsum, w_ref[l],
                            preferred_element_type=jnp.float32)
    return jnp.maximum(acc, 0.0)


_QB = 200
_PB = _QB * NN
_NBLK = N_Q // _QB


def _conv_body(qrep_ref, pos_ref, xg_ref, kpt_ref, w_ref, out_ref):
    rel = pos_ref[...] - qrep_ref[...]
    xg = xg_ref[...].astype(jnp.bfloat16)
    out_ref[...] = _conv_math(rel, kpt_ref, w_ref, xg, _QB)


def _conv_call(qrep, pos, xg3, kpt, weights, interpret=False):
    return pl.pallas_call(
        _conv_body,
        grid=(_NBLK,),
        in_specs=[
            pl.BlockSpec((_PB, 16), lambda i: (i, 0)),
            pl.BlockSpec((_PB, 16), lambda i: (i, 0)),
            pl.BlockSpec((_QB, NN, IN_DIM), lambda i: (i, 0, 0)),
            pl.BlockSpec((16, 32), lambda i: (0, 0)),
            pl.BlockSpec((KL, IN_DIM, OUT_DIM), lambda i: (0, 0, 0)),
        ],
        out_specs=pl.BlockSpec((_QB, OUT_DIM), lambda i: (i, 0)),
        out_shape=jax.ShapeDtypeStruct((N_Q, OUT_DIM), jnp.float32),
        compiler_params=pltpu.CompilerParams(
            dimension_semantics=("arbitrary",)),
        interpret=interpret,
    )(qrep, pos, xg3, kpt, weights)


_QBC = 256
_NBLKC = _CAPQ // _QBC


def _conv_body_c(rel_ref, xg_ref, kpt_ref, w_ref, out_ref):
    rel = rel_ref[...]
    xg = xg_ref[...].astype(jnp.bfloat16)
    out_ref[...] = _conv_math(rel, kpt_ref, w_ref, xg, _QBC)


def _conv_call_c(relc, xgc3, kpt, weights):
    return pl.pallas_call(
        _conv_body_c,
        grid=(_NBLKC,),
        in_specs=[
            pl.BlockSpec((_QBC * NN, 16), lambda i: (i, 0)),
            pl.BlockSpec((_QBC, NN, IN_DIM), lambda i: (i, 0, 0)),
            pl.BlockSpec((16, 32), lambda i: (0, 0)),
            pl.BlockSpec((KL, IN_DIM, OUT_DIM), lambda i: (0, 0, 0)),
        ],
        out_specs=pl.BlockSpec((_QBC, OUT_DIM), lambda i: (i, 0)),
        out_shape=jax.ShapeDtypeStruct((_CAPQ, OUT_DIM), jnp.float32),
        compiler_params=pltpu.CompilerParams(
            dimension_semantics=("arbitrary",)),
    )(relc, xgc3, kpt, weights)


def kernel(q_pts, s_pts, neighb_inds, x, weights, kernel_points):
    idx = (neighb_inds.astype(jnp.int32) % (N_S + 1)).reshape(-1)
    idx_pad = jnp.concatenate(
        [idx, jnp.zeros((ROWS_PAD - ROWS,), jnp.int32)])
    x_pad = jnp.concatenate([x, jnp.zeros((1, IN_DIM), x.dtype)], axis=0)
    tail = jnp.concatenate([jnp.full((1,), 1.0e6, s_pts.dtype),
                            jnp.zeros((_NS_PAD - N_S - 1,), s_pts.dtype)])
    sx = jnp.concatenate([s_pts[:, 0], tail])
    sy = jnp.concatenate([s_pts[:, 1], tail])
    sz = jnp.concatenate([s_pts[:, 2], tail])
    qtail = jnp.full((_NQ_PAD - N_Q,), 1.0e6, q_pts.dtype)
    qx = jnp.concatenate([q_pts[:, 0], qtail])
    qy = jnp.concatenate([q_pts[:, 1], qtail])
    qz = jnp.concatenate([q_pts[:, 2], qtail])
    zz = jnp.zeros((128, 16), jnp.float32)
    kpt = jnp.pad(kernel_points, ((0, 5), (0, 13))).T
    w_bf = weights.astype(jnp.bfloat16)

    xgc, relc, slot_map, cnts = _sc_filter()(
        idx_pad, x_pad, qx, qy, qz, sx, sy, sz, zz)
    overflow = jnp.any(cnts[0::16] > _CAPT)

    def sparse_path(op):
        xgc_, relc_, slot_map_ = op
        fxc = _conv_call_c(relc_, xgc_.reshape(_CAPQ, NN, IN_DIM), kpt, w_bf)
        fxc_pad = jnp.concatenate(
            [fxc, jnp.zeros((8, OUT_DIM), jnp.float32)], axis=0)
        fxp = _sc_scatter()(slot_map_, fxc_pad)
        return fxp[:N_Q]

    def dense_path(op):
        zzc = jnp.zeros((_CH, 16), jnp.float32)
        q16 = jnp.pad(q_pts, ((0, 0), (0, 13)))
        qrep = jnp.broadcast_to(
            q16[:, None, :], (N_Q, NN, 16)).reshape(ROWS, 16)
        xg, pos = _sc_gather()(idx_pad, x_pad, sx, sy, sz, zzc)
        xg3 = xg.reshape(ROWS_PAD // NN, NN, IN_DIM)
        return _conv_call(qrep, pos, xg3, kpt, w_bf)

    return lax.cond(overflow, dense_path, sparse_path, (xgc, relc, slot_map))
